# baseline (speedup 1.0000x reference)
import functools

import jax
import jax.numpy as jnp
from jax import lax
from jax.experimental import pallas as pl
from jax.experimental.pallas import tpu as pltpu
from jax.experimental.pallas import tpu_sc as plsc

_B, _D, _E, _K = 4096, 1024, 8, 2
_TILE = 256
_L = 2 * _B + _E * _TILE
_NT = _L // _TILE
_BT = 1024
_NW = 32
_PPW = 2 * _B // _NW
_TPW = _B // _NW
_CH = 64


def _router_body(x_ref, wg_ref, bg_ref, be_ref,
                 topi_ref, i1_ref, i2_ref, w1_ref, w2_ref, bias_ref):
    xt = x_ref[...]
    logits = jax.lax.dot_general(
        xt, wg_ref[...], (((1,), (1,)), ((), ())),
        preferred_element_type=jnp.float32) + bg_ref[...]
    iota = jax.lax.broadcasted_iota(jnp.int32, logits.shape, 1)
    v1 = jnp.max(logits, axis=1, keepdims=True)
    i1 = jnp.min(jnp.where(logits == v1, iota, _E), axis=1, keepdims=True)
    masked = jnp.where(iota == i1, -jnp.inf, logits)
    v2 = jnp.max(masked, axis=1, keepdims=True)
    i2 = jnp.min(jnp.where(masked == v2, iota, _E), axis=1, keepdims=True)
    t = jnp.exp(v2 - v1)
    denom = 1.0 + t
    w1 = 1.0 / denom
    w2 = t / denom
    comb = (w1 * (iota == i1).astype(jnp.float32)
            + w2 * (iota == i2).astype(jnp.float32))
    bias_ref[...] = jax.lax.dot_general(
        comb, be_ref[...], (((1,), (0,)), ((), ())),
        preferred_element_type=jnp.float32)
    c0 = jnp.zeros(logits.shape, jnp.int32)
    topi_ref[...] = jnp.where(iota == 0, i1, jnp.where(iota == 1, i2, c0))
    i1_ref[...] = i1
    i2_ref[...] = i2
    w1_ref[...] = w1
    w2_ref[...] = w2


def _tables_body(e1_ref, e2_ref, pos0_ref, pos1_ref, te_ref):
    e1 = e1_ref[...]
    e2 = e2_ref[...]
    r_iota = jax.lax.broadcasted_iota(jnp.int32, (128, 128), 0)
    c_iota = jax.lax.broadcasted_iota(jnp.int32, (128, 128), 1)
    ut = (r_iota < c_iota).astype(jnp.float32)
    r32 = jax.lax.broadcasted_iota(jnp.int32, (32, 32), 0)
    c32 = jax.lax.broadcasted_iota(jnp.int32, (32, 32), 1)
    lt = (c32 < r32).astype(jnp.float32)

    cnts = []
    ranks0 = []
    ranks1 = []
    m0s = []
    m1s = []
    for e in range(_E):
        m0 = (e1 == e).astype(jnp.float32)
        m1 = (e2 == e).astype(jnp.float32)
        rp0 = jax.lax.dot_general(m0, ut, (((1,), (0,)), ((), ())),
                                  preferred_element_type=jnp.float32)
        rp1 = jax.lax.dot_general(m1, ut, (((1,), (0,)), ((), ())),
                                  preferred_element_type=jnp.float32)
        tot0 = jnp.sum(m0, axis=1, keepdims=True)
        tot1 = jnp.sum(m1, axis=1, keepdims=True)
        rb0 = jax.lax.dot_general(lt, tot0, (((1,), (0,)), ((), ())),
                                  preferred_element_type=jnp.float32)
        rb1 = jax.lax.dot_general(lt, tot1, (((1,), (0,)), ((), ())),
                                  preferred_element_type=jnp.float32)
        cnt0 = jnp.sum(tot0)
        rank0 = rp0 + rb0
        rank1 = cnt0 + rp1 + rb1
        cnts.append(cnt0 + jnp.sum(tot1))
        ranks0.append(rank0)
        ranks1.append(rank1)
        m0s.append(m0)
        m1s.append(m1)

    start = 0.0
    pos0 = jnp.zeros((32, 128), jnp.float32)
    pos1 = jnp.zeros((32, 128), jnp.float32)
    te_acc = jnp.full((8, 128), -1, jnp.int32)
    tau = (jax.lax.broadcasted_iota(jnp.int32, (8, 128), 0) * 128
           + jax.lax.broadcasted_iota(jnp.int32, (8, 128), 1)).astype(jnp.float32)
    for e in range(_E):
        pos0 = pos0 + m0s[e] * (start + ranks0[e])
        pos1 = pos1 + m1s[e] * (start + ranks1[e])
        te_acc = te_acc + (tau * float(_TILE) >= start).astype(jnp.int32)
        padded = jnp.ceil(cnts[e] / _TILE) * _TILE
        start = start + padded
    pos0_ref[...] = pos0.astype(jnp.int32)
    pos1_ref[...] = pos1.astype(jnp.int32)
    te_ref[...] = jnp.clip(te_acc, 0, _E - 1)


def _matmul_body(te_ref, xg_ref, we_ref, y_ref):
    del te_ref
    y = jax.lax.dot_general(
        xg_ref[...].astype(jnp.bfloat16), we_ref[0],
        (((1,), (1,)), ((), ())),
        preferred_element_type=jnp.float32)
    y_ref[...] = y


def _combine_body(bias_ref, w1_ref, w2_ref, y0_ref, y1_ref, out_ref):
    out_ref[...] = (bias_ref[...]
                    + w1_ref[...] * y0_ref[...]
                    + w2_ref[...] * y1_ref[...])


def _sc_dispatch_body(xb_hbm, pos_hbm, xg_hbm, idx_v, rows_v, sem):
    nc = 2
    wid = lax.axis_index("s") * nc + lax.axis_index("c")
    base = wid * _PPW

    def chunk(i, _):
        off = base + i * _CH
        toff = lax.rem(off, _B)
        pltpu.sync_copy(xb_hbm.at[pl.ds(toff, _CH)], rows_v)
        pltpu.sync_copy(pos_hbm.at[pl.ds(off, _CH)], idx_v)
        pltpu.async_copy(rows_v, xg_hbm.at[idx_v], sem).wait()
        return 0

    lax.fori_loop(0, _PPW // _CH, chunk, 0)


def _sc_unpermute_body(yp_hbm, pos0_hbm, pos1_hbm, y0_hbm, y1_hbm,
                       idx0_v, idx1_v, r0_v, sem):
    nc = 2
    wid = lax.axis_index("s") * nc + lax.axis_index("c")
    base = wid * _TPW

    def chunk(i, _):
        off = base + i * _CH
        pltpu.sync_copy(pos0_hbm.at[pl.ds(off, _CH)], idx0_v)
        pltpu.async_copy(yp_hbm.at[idx0_v], r0_v, sem).wait()
        pltpu.sync_copy(r0_v, y0_hbm.at[pl.ds(off, _CH)])
        pltpu.sync_copy(pos1_hbm.at[pl.ds(off, _CH)], idx1_v)
        pltpu.async_copy(yp_hbm.at[idx1_v], r0_v, sem).wait()
        pltpu.sync_copy(r0_v, y1_hbm.at[pl.ds(off, _CH)])
        return 0

    lax.fori_loop(0, _TPW // _CH, chunk, 0)


def _sc_dispatch(xb, pos):
    mesh = plsc.VectorSubcoreMesh(core_axis_name="c", subcore_axis_name="s")
    return pl.kernel(
        _sc_dispatch_body,
        mesh=mesh,
        out_type=jax.ShapeDtypeStruct((_L, _D), jnp.float32),
        scratch_types=[
            pltpu.VMEM((_CH,), jnp.int32),
            pltpu.VMEM((_CH, _D), jnp.float32),
            pltpu.SemaphoreType.DMA,
        ],
    )(xb, pos)


def _sc_unpermute(yperm, pos0, pos1):
    mesh = plsc.VectorSubcoreMesh(core_axis_name="c", subcore_axis_name="s")
    return pl.kernel(
        _sc_unpermute_body,
        mesh=mesh,
        out_type=[
            jax.ShapeDtypeStruct((_B, _D), jnp.float32),
            jax.ShapeDtypeStruct((_B, _D), jnp.float32),
        ],
        scratch_types=[
            pltpu.VMEM((_CH,), jnp.int32),
            pltpu.VMEM((_CH,), jnp.int32),
            pltpu.VMEM((_CH, _D), jnp.float32),
            pltpu.SemaphoreType.DMA,
        ],
    )(yperm, pos0, pos1)


def _router(x, Wg, bg, be):
    nt = _B // _BT
    return pl.pallas_call(
        _router_body,
        grid=(nt,),
        in_specs=[
            pl.BlockSpec((_BT, _D), lambda i: (i, 0)),
            pl.BlockSpec((_E, _D), lambda i: (0, 0)),
            pl.BlockSpec((1, _E), lambda i: (0, 0)),
            pl.BlockSpec((_E, _D), lambda i: (0, 0)),
        ],
        out_specs=[
            pl.BlockSpec((_BT, _E), lambda i: (i, 0)),
            pl.BlockSpec((_BT, 1), lambda i: (i, 0)),
            pl.BlockSpec((_BT, 1), lambda i: (i, 0)),
            pl.BlockSpec((_BT, 1), lambda i: (i, 0)),
            pl.BlockSpec((_BT, 1), lambda i: (i, 0)),
            pl.BlockSpec((_BT, _D), lambda i: (i, 0)),
        ],
        out_shape=[
            jax.ShapeDtypeStruct((_B, _E), jnp.int32),
            jax.ShapeDtypeStruct((_B, 1), jnp.int32),
            jax.ShapeDtypeStruct((_B, 1), jnp.int32),
            jax.ShapeDtypeStruct((_B, 1), jnp.float32),
            jax.ShapeDtypeStruct((_B, 1), jnp.float32),
            jax.ShapeDtypeStruct((_B, _D), jnp.float32),
        ],
    )(x, Wg, bg.reshape(1, _E), be)


def _tables(e1, e2):
    return pl.pallas_call(
        _tables_body,
        grid=(1,),
        in_specs=[
            pl.BlockSpec((32, 128), lambda i: (0, 0)),
            pl.BlockSpec((32, 128), lambda i: (0, 0)),
        ],
        out_specs=[
            pl.BlockSpec((32, 128), lambda i: (0, 0)),
            pl.BlockSpec((32, 128), lambda i: (0, 0)),
            pl.BlockSpec((8, 128), lambda i: (0, 0)),
        ],
        out_shape=[
            jax.ShapeDtypeStruct((32, 128), jnp.int32),
            jax.ShapeDtypeStruct((32, 128), jnp.int32),
            jax.ShapeDtypeStruct((8, 128), jnp.int32),
        ],
    )(e1, e2)


def _grouped_matmul(te, xg, Web):
    grid_spec = pltpu.PrefetchScalarGridSpec(
        num_scalar_prefetch=1,
        grid=(_NT,),
        in_specs=[
            pl.BlockSpec((_TILE, _D), lambda i, te_ref: (i, 0)),
            pl.BlockSpec((1, _D, _D), lambda i, te_ref: (te_ref[i], 0, 0)),
        ],
        out_specs=pl.BlockSpec((_TILE, _D), lambda i, te_ref: (i, 0)),
    )
    return pl.pallas_call(
        _matmul_body,
        grid_spec=grid_spec,
        out_shape=jax.ShapeDtypeStruct((_L, _D), jnp.float32),
    )(te, xg, Web)


def _combine(bias_part, w1, w2, y0, y1):
    nt = _B // _BT
    return pl.pallas_call(
        _combine_body,
        grid=(nt,),
        in_specs=[
            pl.BlockSpec((_BT, _D), lambda i: (i, 0)),
            pl.BlockSpec((_BT, 1), lambda i: (i, 0)),
            pl.BlockSpec((_BT, 1), lambda i: (i, 0)),
            pl.BlockSpec((_BT, _D), lambda i: (i, 0)),
            pl.BlockSpec((_BT, _D), lambda i: (i, 0)),
        ],
        out_specs=pl.BlockSpec((_BT, _D), lambda i: (i, 0)),
        out_shape=jax.ShapeDtypeStruct((_B, _D), jnp.float32),
    )(bias_part, w1, w2, y0, y1)


@jax.jit
def _moe(x, Wg, bg, We, be):
    topi, i1, i2, w1, w2, bias_part = _router(x, Wg, bg, be)
    e1 = i1.reshape(32, 128)
    e2 = i2.reshape(32, 128)
    pos0_2d, pos1_2d, te_pad = _tables(e1, e2)
    pos0 = pos0_2d.reshape(_B)
    pos1 = pos1_2d.reshape(_B)
    pos = jnp.concatenate([pos0, pos1])
    te = te_pad.reshape(-1)[:_NT]
    xg = _sc_dispatch(x, pos)
    Web = We.astype(jnp.bfloat16)
    yperm = _grouped_matmul(te, xg, Web)
    y0, y1 = _sc_unpermute(yperm, pos0, pos1)
    out = _combine(bias_part, w1, w2, y0, y1)
    return out, topi[:, :_K]


def kernel(x, Wg, bg, We, be):
    return _moe(x, Wg, bg, We, be)


# speedup vs baseline: 1.3063x; 1.3063x over previous
import functools

import jax
import jax.numpy as jnp
from jax import lax
from jax.experimental import pallas as pl
from jax.experimental.pallas import tpu as pltpu
from jax.experimental.pallas import tpu_sc as plsc

_B, _D, _E, _K = 4096, 1024, 8, 2
_TILE = 256
_L = 2 * _B + _E * _TILE
_NT = _L // _TILE
_BT = 1024
_NW = 32
_PPW = 2 * _B // _NW
_TPW = _B // _NW
_CH = 64


H = _D // 2


def _pack_bf16(y):
    yb = y.astype(jnp.bfloat16)
    lo = jax.lax.bitcast_convert_type(yb[:, :H], jnp.int16).astype(jnp.int32)
    hi = jax.lax.bitcast_convert_type(yb[:, H:], jnp.int16).astype(jnp.int32)
    return (lo & 0xFFFF) | (hi << 16)


def _unpack_bf16(p):
    lo = jax.lax.bitcast_convert_type(p.astype(jnp.int16), jnp.bfloat16)
    hi = jax.lax.bitcast_convert_type(
        (p >> 16).astype(jnp.int16), jnp.bfloat16)
    return jnp.concatenate([lo, hi], axis=1)


def _router_body(x_ref, wg_ref, bg_ref, be_ref,
                 topi_ref, i1_ref, i2_ref, w1_ref, w2_ref, bias_ref, xb_ref):
    xt = x_ref[...]
    logits = jax.lax.dot_general(
        xt, wg_ref[...], (((1,), (1,)), ((), ())),
        preferred_element_type=jnp.float32) + bg_ref[...]
    iota = jax.lax.broadcasted_iota(jnp.int32, logits.shape, 1)
    v1 = jnp.max(logits, axis=1, keepdims=True)
    i1 = jnp.min(jnp.where(logits == v1, iota, _E), axis=1, keepdims=True)
    masked = jnp.where(iota == i1, -jnp.inf, logits)
    v2 = jnp.max(masked, axis=1, keepdims=True)
    i2 = jnp.min(jnp.where(masked == v2, iota, _E), axis=1, keepdims=True)
    t = jnp.exp(v2 - v1)
    denom = 1.0 + t
    w1 = 1.0 / denom
    w2 = t / denom
    comb = (w1 * (iota == i1).astype(jnp.float32)
            + w2 * (iota == i2).astype(jnp.float32))
    bias_ref[...] = jax.lax.dot_general(
        comb, be_ref[...], (((1,), (0,)), ((), ())),
        preferred_element_type=jnp.float32)
    xb_ref[...] = _pack_bf16(xt)
    c0 = jnp.zeros(logits.shape, jnp.int32)
    topi_ref[...] = jnp.where(iota == 0, i1, jnp.where(iota == 1, i2, c0))
    i1_ref[...] = i1
    i2_ref[...] = i2
    w1_ref[...] = w1
    w2_ref[...] = w2


def _tables_body(e1_ref, e2_ref, pos0_ref, pos1_ref, te_ref):
    e1 = e1_ref[...]
    e2 = e2_ref[...]
    r_iota = jax.lax.broadcasted_iota(jnp.int32, (128, 128), 0)
    c_iota = jax.lax.broadcasted_iota(jnp.int32, (128, 128), 1)
    ut = (r_iota < c_iota).astype(jnp.float32)
    r32 = jax.lax.broadcasted_iota(jnp.int32, (32, 32), 0)
    c32 = jax.lax.broadcasted_iota(jnp.int32, (32, 32), 1)
    lt = (c32 < r32).astype(jnp.float32)

    cnts = []
    ranks0 = []
    ranks1 = []
    m0s = []
    m1s = []
    for e in range(_E):
        m0 = (e1 == e).astype(jnp.float32)
        m1 = (e2 == e).astype(jnp.float32)
        rp0 = jax.lax.dot_general(m0, ut, (((1,), (0,)), ((), ())),
                                  preferred_element_type=jnp.float32)
        rp1 = jax.lax.dot_general(m1, ut, (((1,), (0,)), ((), ())),
                                  preferred_element_type=jnp.float32)
        tot0 = jnp.sum(m0, axis=1, keepdims=True)
        tot1 = jnp.sum(m1, axis=1, keepdims=True)
        rb0 = jax.lax.dot_general(lt, tot0, (((1,), (0,)), ((), ())),
                                  preferred_element_type=jnp.float32)
        rb1 = jax.lax.dot_general(lt, tot1, (((1,), (0,)), ((), ())),
                                  preferred_element_type=jnp.float32)
        cnt0 = jnp.sum(tot0)
        rank0 = rp0 + rb0
        rank1 = cnt0 + rp1 + rb1
        cnts.append(cnt0 + jnp.sum(tot1))
        ranks0.append(rank0)
        ranks1.append(rank1)
        m0s.append(m0)
        m1s.append(m1)

    start = 0.0
    pos0 = jnp.zeros((32, 128), jnp.float32)
    pos1 = jnp.zeros((32, 128), jnp.float32)
    te_acc = jnp.full((8, 128), -1, jnp.int32)
    tau = (jax.lax.broadcasted_iota(jnp.int32, (8, 128), 0) * 128
           + jax.lax.broadcasted_iota(jnp.int32, (8, 128), 1)).astype(jnp.float32)
    for e in range(_E):
        pos0 = pos0 + m0s[e] * (start + ranks0[e])
        pos1 = pos1 + m1s[e] * (start + ranks1[e])
        te_acc = te_acc + (tau * float(_TILE) >= start).astype(jnp.int32)
        padded = jnp.ceil(cnts[e] / _TILE) * _TILE
        start = start + padded
    pos0_ref[...] = pos0.astype(jnp.int32)
    pos1_ref[...] = pos1.astype(jnp.int32)
    te_ref[...] = jnp.clip(te_acc, 0, _E - 1)


def _matmul_body(te_ref, xg_ref, we_ref, y_ref):
    del te_ref
    xb = _unpack_bf16(xg_ref[...])
    y = jax.lax.dot_general(
        xb, we_ref[0].astype(jnp.bfloat16), (((1,), (1,)), ((), ())),
        preferred_element_type=jnp.float32)
    y_ref[...] = _pack_bf16(y)


def _combine_body(bias_ref, w1_ref, w2_ref, y0_ref, y1_ref, out_ref):
    y0 = _unpack_bf16(y0_ref[...]).astype(jnp.float32)
    y1 = _unpack_bf16(y1_ref[...]).astype(jnp.float32)
    out_ref[...] = (bias_ref[...]
                    + w1_ref[...] * y0
                    + w2_ref[...] * y1)


_CH2 = 64


def _sc_dispatch_body(xb_hbm, pos0_hbm, pos1_hbm, xg_hbm,
                      idx0, idx1, rows0, rows1, l0, l1, s0, s1):
    nc = 2
    wid = lax.axis_index("s") * nc + lax.axis_index("c")
    base = wid * (_B // _NW)
    nch = _PPW // _CH2
    idxs = (idx0, idx1)
    rows = (rows0, rows1)
    lsems = (l0, l1)
    ssems = (s0, s1)

    def load(i):
        b = i % 2
        slot_hbm = pos0_hbm if i < nch // 2 else pos1_hbm
        toff = base + (i % (nch // 2)) * _CH2
        ca = pltpu.async_copy(xb_hbm.at[pl.ds(toff, _CH2)], rows[b], lsems[b])
        cb = pltpu.async_copy(slot_hbm.at[pl.ds(toff, _CH2)], idxs[b], lsems[b])
        return ca, cb

    loads = [None] * nch
    scat = [None] * nch
    loads[0] = load(0)
    for i in range(nch):
        b = i % 2
        loads[i][0].wait()
        loads[i][1].wait()
        scat[i] = pltpu.async_copy(rows[b], xg_hbm.at[idxs[b]], ssems[b])
        if i + 1 < nch:
            if i - 1 >= 0:
                scat[i - 1].wait()
            loads[i + 1] = load(i + 1)
    scat[nch - 1].wait()
    if nch >= 2:
        scat[nch - 2].wait()


def _sc_unpermute_body(yp_hbm, pos0_hbm, pos1_hbm, y0_hbm, y1_hbm,
                       idx0_v, idx1_v, r0_v, r1_v, lsem, g0, g1, w0, w1):
    nc = 2
    wid = lax.axis_index("s") * nc + lax.axis_index("c")
    base = wid * _TPW
    nj = 2 * (_TPW // _CH2)
    rows = (r0_v, r1_v)
    gsems = (g0, g1)
    wsems = (w0, w1)

    ca = pltpu.async_copy(pos0_hbm.at[pl.ds(base, _TPW)], idx0_v, lsem)
    cb = pltpu.async_copy(pos1_hbm.at[pl.ds(base, _TPW)], idx1_v, lsem)
    ca.wait()
    cb.wait()

    def gather(j):
        b = j % 2
        chunkpos = j // 2
        idx_full = idx0_v if (j & 1) == 0 else idx1_v
        return pltpu.async_copy(
            yp_hbm.at[idx_full.at[pl.ds(chunkpos * _CH2, _CH2)]],
            rows[b], gsems[b])

    def store(j):
        b = j % 2
        chunkpos = j // 2
        off = base + chunkpos * _CH2
        yout = y0_hbm if (j & 1) == 0 else y1_hbm
        return pltpu.async_copy(rows[b], yout.at[pl.ds(off, _CH2)], wsems[b])

    g = [None] * nj
    wr = [None] * nj
    g[0] = gather(0)
    for j in range(nj):
        g[j].wait()
        if j + 1 < nj:
            if j - 1 >= 0:
                wr[j - 1].wait()
            g[j + 1] = gather(j + 1)
        wr[j] = store(j)
    wr[nj - 1].wait()
    if nj >= 2:
        wr[nj - 2].wait()


def _sc_dispatch(xb, pos0, pos1):
    mesh = plsc.VectorSubcoreMesh(core_axis_name="c", subcore_axis_name="s")
    return pl.kernel(
        _sc_dispatch_body,
        mesh=mesh,
        out_type=jax.ShapeDtypeStruct((_L, _D // 2), jnp.int32),
        scratch_types=[
            pltpu.VMEM((_CH2,), jnp.int32),
            pltpu.VMEM((_CH2,), jnp.int32),
            pltpu.VMEM((_CH2, _D // 2), jnp.int32),
            pltpu.VMEM((_CH2, _D // 2), jnp.int32),
            pltpu.SemaphoreType.DMA,
            pltpu.SemaphoreType.DMA,
            pltpu.SemaphoreType.DMA,
            pltpu.SemaphoreType.DMA,
        ],
    )(xb, pos0, pos1)


def _sc_unpermute(yperm, pos0, pos1):
    mesh = plsc.VectorSubcoreMesh(core_axis_name="c", subcore_axis_name="s")
    return pl.kernel(
        _sc_unpermute_body,
        mesh=mesh,
        out_type=[
            jax.ShapeDtypeStruct((_B, _D // 2), jnp.int32),
            jax.ShapeDtypeStruct((_B, _D // 2), jnp.int32),
        ],
        scratch_types=[
            pltpu.VMEM((_TPW,), jnp.int32),
            pltpu.VMEM((_TPW,), jnp.int32),
            pltpu.VMEM((_CH2, _D // 2), jnp.int32),
            pltpu.VMEM((_CH2, _D // 2), jnp.int32),
            pltpu.SemaphoreType.DMA,
            pltpu.SemaphoreType.DMA,
            pltpu.SemaphoreType.DMA,
            pltpu.SemaphoreType.DMA,
            pltpu.SemaphoreType.DMA,
        ],
    )(yperm, pos0, pos1)


def _router(x, Wg, bg, be):
    nt = _B // _BT
    return pl.pallas_call(
        _router_body,
        grid=(nt,),
        in_specs=[
            pl.BlockSpec((_BT, _D), lambda i: (i, 0)),
            pl.BlockSpec((_E, _D), lambda i: (0, 0)),
            pl.BlockSpec((1, _E), lambda i: (0, 0)),
            pl.BlockSpec((_E, _D), lambda i: (0, 0)),
        ],
        out_specs=[
            pl.BlockSpec((_BT, _E), lambda i: (i, 0)),
            pl.BlockSpec((_BT, 1), lambda i: (i, 0)),
            pl.BlockSpec((_BT, 1), lambda i: (i, 0)),
            pl.BlockSpec((_BT, 1), lambda i: (i, 0)),
            pl.BlockSpec((_BT, 1), lambda i: (i, 0)),
            pl.BlockSpec((_BT, _D), lambda i: (i, 0)),
            pl.BlockSpec((_BT, _D // 2), lambda i: (i, 0)),
        ],
        out_shape=[
            jax.ShapeDtypeStruct((_B, _E), jnp.int32),
            jax.ShapeDtypeStruct((_B, 1), jnp.int32),
            jax.ShapeDtypeStruct((_B, 1), jnp.int32),
            jax.ShapeDtypeStruct((_B, 1), jnp.float32),
            jax.ShapeDtypeStruct((_B, 1), jnp.float32),
            jax.ShapeDtypeStruct((_B, _D), jnp.float32),
            jax.ShapeDtypeStruct((_B, _D // 2), jnp.int32),
        ],
    )(x, Wg, bg.reshape(1, _E), be)


def _tables(e1, e2):
    return pl.pallas_call(
        _tables_body,
        grid=(1,),
        in_specs=[
            pl.BlockSpec((32, 128), lambda i: (0, 0)),
            pl.BlockSpec((32, 128), lambda i: (0, 0)),
        ],
        out_specs=[
            pl.BlockSpec((32, 128), lambda i: (0, 0)),
            pl.BlockSpec((32, 128), lambda i: (0, 0)),
            pl.BlockSpec((8, 128), lambda i: (0, 0)),
        ],
        out_shape=[
            jax.ShapeDtypeStruct((32, 128), jnp.int32),
            jax.ShapeDtypeStruct((32, 128), jnp.int32),
            jax.ShapeDtypeStruct((8, 128), jnp.int32),
        ],
    )(e1, e2)


def _grouped_matmul(te, xg, Web):
    grid_spec = pltpu.PrefetchScalarGridSpec(
        num_scalar_prefetch=1,
        grid=(_NT,),
        in_specs=[
            pl.BlockSpec((_TILE, _D // 2), lambda i, te_ref: (i, 0)),
            pl.BlockSpec((1, _D, _D), lambda i, te_ref: (te_ref[i], 0, 0)),
        ],
        out_specs=pl.BlockSpec((_TILE, _D // 2), lambda i, te_ref: (i, 0)),
    )
    return pl.pallas_call(
        _matmul_body,
        grid_spec=grid_spec,
        out_shape=jax.ShapeDtypeStruct((_L, _D // 2), jnp.int32),
    )(te, xg, Web)


def _combine(bias_part, w1, w2, y0, y1):
    nt = _B // _BT
    return pl.pallas_call(
        _combine_body,
        grid=(nt,),
        in_specs=[
            pl.BlockSpec((_BT, _D), lambda i: (i, 0)),
            pl.BlockSpec((_BT, 1), lambda i: (i, 0)),
            pl.BlockSpec((_BT, 1), lambda i: (i, 0)),
            pl.BlockSpec((_BT, _D // 2), lambda i: (i, 0)),
            pl.BlockSpec((_BT, _D // 2), lambda i: (i, 0)),
        ],
        out_specs=pl.BlockSpec((_BT, _D), lambda i: (i, 0)),
        out_shape=jax.ShapeDtypeStruct((_B, _D), jnp.float32),
    )(bias_part, w1, w2, y0, y1)


@jax.jit
def _moe(x, Wg, bg, We, be):
    topi, i1, i2, w1, w2, bias_part, xb = _router(x, Wg, bg, be)
    e1 = i1.reshape(32, 128)
    e2 = i2.reshape(32, 128)
    pos0_2d, pos1_2d, te_pad = _tables(e1, e2)
    pos0 = pos0_2d.reshape(_B)
    pos1 = pos1_2d.reshape(_B)
    te = te_pad.reshape(-1)[:_NT]
    xg = _sc_dispatch(xb, pos0, pos1)
    yperm = _grouped_matmul(te, xg, We)
    y0, y1 = _sc_unpermute(yperm, pos0, pos1)
    out = _combine(bias_part, w1, w2, y0, y1)
    return out, topi[:, :_K]


def kernel(x, Wg, bg, We, be):
    return _moe(x, Wg, bg, We, be)


# speedup vs baseline: 1.8555x; 1.4205x over previous
import functools

import jax
import jax.numpy as jnp
from jax.experimental import pallas as pl


_B, _D, _E, _K = 4096, 1024, 8, 2
_BT = 1024


def _fused_moe_body(x_ref, wg_ref, bg_ref, we_ref, be_ref, out_ref, top_ref):
    xt = x_ref[...]
    logits = jax.lax.dot_general(
        xt, wg_ref[...], (((1,), (1,)), ((), ())),
        preferred_element_type=jnp.float32) + bg_ref[...]
    iota = jax.lax.broadcasted_iota(jnp.int32, logits.shape, 1)
    v1 = jnp.max(logits, axis=1, keepdims=True)
    i1 = jnp.min(jnp.where(logits == v1, iota, _E), axis=1, keepdims=True)
    masked = jnp.where(iota == i1, -jnp.inf, logits)
    v2 = jnp.max(masked, axis=1, keepdims=True)
    i2 = jnp.min(jnp.where(masked == v2, iota, _E), axis=1, keepdims=True)
    t = jnp.exp(v2 - v1)
    denom = 1.0 + t
    w1 = 1.0 / denom
    w2 = t / denom
    comb = (w1 * (iota == i1).astype(jnp.float32)
            + w2 * (iota == i2).astype(jnp.float32))

    acc = jax.lax.dot_general(
        comb, be_ref[...], (((1,), (0,)), ((), ())),
        preferred_element_type=jnp.float32)
    xb = xt.astype(jnp.bfloat16)
    for e in range(_E):
        ye = jax.lax.dot_general(
            xb, we_ref[e], (((1,), (1,)), ((), ())),
            preferred_element_type=jnp.float32)
        acc = acc + comb[:, e:e + 1] * ye
    out_ref[...] = acc
    c0 = jnp.zeros(logits.shape, jnp.int32)
    top_ref[...] = jnp.where(iota == 0, i1, jnp.where(iota == 1, i2, c0))


@jax.jit
def _fused_moe(x, Wg, bg, We, be):
    nt = _B // _BT
    out, top_pad = pl.pallas_call(
        _fused_moe_body,
        grid=(nt,),
        in_specs=[
            pl.BlockSpec((_BT, _D), lambda i: (i, 0)),
            pl.BlockSpec((_E, _D), lambda i: (0, 0)),
            pl.BlockSpec((1, _E), lambda i: (0, 0)),
            pl.BlockSpec((_E, _D, _D), lambda i: (0, 0, 0)),
            pl.BlockSpec((_E, _D), lambda i: (0, 0)),
        ],
        out_specs=[
            pl.BlockSpec((_BT, _D), lambda i: (i, 0)),
            pl.BlockSpec((_BT, _E), lambda i: (i, 0)),
        ],
        out_shape=[
            jax.ShapeDtypeStruct((_B, _D), jnp.float32),
            jax.ShapeDtypeStruct((_B, _E), jnp.int32),
        ],
    )(x, Wg, bg.reshape(1, _E), We.astype(jnp.bfloat16), be)
    return out, top_pad[:, :_K]


def kernel(x, Wg, bg, We, be):
    return _fused_moe(x, Wg, bg, We, be)
